# baseline (speedup 1.0000x reference)
import functools

import jax
import jax.numpy as jnp
from jax import lax
from jax.experimental import pallas as pl
from jax.experimental.pallas import tpu as pltpu
from jax.experimental.pallas import tpu_sc as plsc

_NUM_EMBEDDINGS = 1000000
_EMBED_DIM = 64
_BATCH = 16384

_NC = 2
_NS = 16
_NW = _NC * _NS
_BPW = _BATCH // _NW
_CHUNK = 128
_NCHUNK = _BPW // _CHUNK


@functools.partial(
    pl.kernel,
    mesh=plsc.VectorSubcoreMesh(core_axis_name="c", subcore_axis_name="s"),
    compiler_params=pltpu.CompilerParams(use_tc_tiling_on_sc=False),
    out_type=jax.ShapeDtypeStruct((_NW, _NCHUNK, _CHUNK, _EMBED_DIM),
                                  jnp.float32),
    scratch_types=[
        pltpu.VMEM((_NCHUNK, _CHUNK), jnp.int32),
        pltpu.VMEM((_NCHUNK, _CHUNK, _EMBED_DIM), jnp.float32),
        pltpu.SemaphoreType.DMA,
    ],
)
def _gather_rows(idx_hbm, table_hbm, out_hbm, idx_v, rows_v, sem):
    wid = lax.axis_index("s") * _NC + lax.axis_index("c")
    pltpu.sync_copy(idx_hbm.at[wid], idx_v)
    copies = [
        pltpu.async_copy(table_hbm.at[idx_v.at[j]], rows_v.at[j], sem)
        for j in range(_NCHUNK)
    ]
    for c in copies:
        c.wait()
    pltpu.sync_copy(rows_v, out_hbm.at[wid])


def kernel(input, table):
    idx = input.astype(jnp.int32).reshape(_NW, _NCHUNK, _CHUNK)
    rows = _gather_rows(idx, table)
    return rows.reshape(_BATCH, 1, 8, 8)


# speedup vs baseline: 2.5355x; 2.5355x over previous
import functools

import jax
import jax.numpy as jnp
from jax import lax
from jax.experimental import pallas as pl
from jax.experimental.pallas import tpu as pltpu
from jax.experimental.pallas import tpu_sc as plsc

_NUM_EMBEDDINGS = 1000000
_EMBED_DIM = 64
_BATCH = 16384

_NC = 2
_NS = 16
_NW = _NC * _NS
_BPW = _BATCH // _NW
_RING = 4
_L = 16
_NBLK = _BPW // _L


def _extract_scalar(vec, lane_onehot):
    return jnp.sum(jnp.where(lane_onehot, vec, 0))


@functools.partial(
    pl.kernel,
    mesh=plsc.VectorSubcoreMesh(core_axis_name="c", subcore_axis_name="s"),
    compiler_params=pltpu.CompilerParams(needs_layout_passes=False),
    out_type=jax.ShapeDtypeStruct((_BATCH, 2 * _EMBED_DIM), jnp.float32),
    scratch_types=[
        pltpu.VMEM((_BATCH,), jnp.int32),
        pltpu.VMEM((_RING, _EMBED_DIM, 128), jnp.float32),
        pltpu.VMEM((_L, 2 * _EMBED_DIM), jnp.float32),
        pltpu.SemaphoreType.DMA,
        pltpu.SemaphoreType.DMA,
        pltpu.SemaphoreType.DMA,
        pltpu.SemaphoreType.DMA,
    ],
)
def _gather_cols(idx_hbm, tab_hbm, out_hbm, idx_v, bufs_v, stage_v, *sems):
    wid = lax.axis_index("s") * _NC + lax.axis_index("c")
    base = wid * _BPW
    pltpu.sync_copy(idx_hbm, idx_v)
    iota = lax.iota(jnp.int32, _L)

    def _fetch(chunk, lane, slot):
        i_s = _extract_scalar(chunk, iota == lane)
        c0 = pl.multiple_of((i_s >> 7) * 128, 128)
        pltpu.async_copy(
            tab_hbm.at[:, pl.ds(c0, 128)], bufs_v.at[slot], sems[slot])

    chunk0 = idx_v[pl.ds(base, _L)]
    for r in range(_RING):
        _fetch(chunk0, r, r)

    def _blk_body(blk, carry):
        chunk = idx_v[pl.ds(base + blk * _L, _L)]
        nxt_blk = jnp.minimum(blk + 1, _NBLK - 1)
        nxt = idx_v[pl.ds(base + nxt_blk * _L, _L)]
        for l in range(_L):
            slot = l % _RING
            pltpu.make_async_copy(
                tab_hbm.at[:, pl.ds(0, 128)], bufs_v.at[slot], sems[slot]
            ).wait()
            lcol = _extract_scalar(chunk, iota == l) & 127
            col_idx = jnp.broadcast_to(lcol, (_L,))
            for j in range(_EMBED_DIM // _L):
                vals = plsc.load_gather(
                    bufs_v.at[slot], [iota + j * _L, col_idx])
                stage_v[l, pl.ds(j * _L, _L)] = vals
            if l < _L - _RING:
                _fetch(chunk, l + _RING, slot)
            else:
                @pl.when(blk < _NBLK - 1)
                def _():
                    _fetch(nxt, l + _RING - _L, slot)
        pltpu.sync_copy(
            stage_v, out_hbm.at[pl.ds(base + blk * _L, _L)])
        return carry

    lax.fori_loop(0, _NBLK, _blk_body, 0)


def kernel(input, table):
    idx = input.astype(jnp.int32)
    out_p = _gather_cols(idx, table.T)
    return out_p[:, :_EMBED_DIM].reshape(_BATCH, 1, 8, 8)
